# baseline (speedup 1.0000x reference)
import functools
import jax
import jax.numpy as jnp
from jax import lax
from jax.experimental import pallas as pl
from jax.experimental.pallas import tpu as pltpu
from jax.experimental.pallas import tpu_sc as plsc

NC, NS = 2, 16
NW = NC * NS
R_SC = 16


def _make_sc(batch, rows_total, dim, row_offset, out_rows):
    rows_per_w = rows_total // NW
    nchunk = rows_per_w // R_SC
    assert nchunk % 2 == 0 and nchunk >= 2
    nvec = dim // 16
    mesh = plsc.VectorSubcoreMesh(core_axis_name="c", subcore_axis_name="s")

    @functools.partial(
        pl.kernel,
        mesh=mesh,
        out_type=jax.ShapeDtypeStruct((batch, out_rows, dim), jnp.float32),
        scratch_types=[
            pltpu.VMEM((16,), jnp.float32),
            pltpu.VMEM((R_SC, dim), jnp.float32),
            pltpu.VMEM((R_SC, dim), jnp.float32),
            pltpu.VMEM((R_SC, dim), jnp.float32),
            pltpu.VMEM((R_SC, dim), jnp.float32),
            pltpu.VMEM((R_SC, dim), jnp.float32),
            pltpu.VMEM((R_SC, dim), jnp.float32),
            pltpu.SemaphoreType.DMA((2,)),
            pltpu.SemaphoreType.DMA((2,)),
            pltpu.SemaphoreType.DMA((2,)),
        ],
    )
    def sc_fn(in_hbm, pe_hbm, scale_hbm, out_hbm,
              scale_v, pe_v0, pe_v1, in_v0, in_v1, out_v0, out_v1,
              pe_sem, in_sem, out_sem):
        wid = lax.axis_index("s") * NC + lax.axis_index("c")
        base = wid * rows_per_w
        pe_bufs = (pe_v0, pe_v1)
        in_bufs = (in_v0, in_v1)
        out_bufs = (out_v0, out_v1)

        def pe_copy(c, p):
            return pltpu.make_async_copy(
                pe_hbm.at[pl.ds(row_offset + base + c * R_SC, R_SC), :],
                pe_bufs[p], pe_sem.at[p])

        def in_copy(c, b, sl):
            return pltpu.make_async_copy(
                in_hbm.at[b, pl.ds(row_offset + base + c * R_SC, R_SC), :],
                in_bufs[sl], in_sem.at[sl])

        def out_copy(c, b, sl):
            return pltpu.make_async_copy(
                out_bufs[sl],
                out_hbm.at[b, pl.ds(base + c * R_SC, R_SC), :],
                out_sem.at[sl])

        pltpu.sync_copy(scale_hbm, scale_v)
        s = scale_v[...]

        pe_copy(0, 0).start()
        in_copy(0, 0, 0).start()

        def compute(in_v, pe_v, out_v):
            def row_body(r, _):
                for j in range(nvec):
                    sl = pl.ds(j * 16, 16)
                    out_v[r, sl] = in_v[r, sl] + pe_v[r, sl] * s
                return 0
            lax.fori_loop(0, R_SC, row_body, 0)

        def half(c2, half_idx):
            c = 2 * c2 + half_idx
            pe_copy(c, half_idx).wait()
            for b in range(batch):
                slot = b % 2
                in_copy(c, b, slot).wait()
                if b + 1 < batch:
                    in_copy(c, b + 1, (b + 1) % 2).start()
                elif half_idx == 0:
                    in_copy(c + 1, 0, 0).start()
                else:
                    @pl.when(c + 1 < nchunk)
                    def _():
                        in_copy(c + 1, 0, 0).start()
                if b == 0:
                    if half_idx == 0:
                        pe_copy(c + 1, 1).start()
                    else:
                        @pl.when(c + 1 < nchunk)
                        def _():
                            pe_copy(c + 1, 0).start()
                k = half_idx * batch + b
                if k >= 2:
                    out_copy(0, 0, slot).wait()
                else:
                    @pl.when(c2 > 0)
                    def _():
                        out_copy(0, 0, slot).wait()
                compute(in_bufs[slot], pe_bufs[half_idx], out_bufs[slot])
                out_copy(c, b, slot).start()

        def c2_body(c2, _):
            half(c2, 0)
            half(c2, 1)
            return 0

        lax.fori_loop(0, nchunk // 2, c2_body, 0)

        out_copy(0, 0, 0).wait()
        out_copy(0, 0, 1).wait()

    return sc_fn


def kernel(input, pe, scale_param):
    batch, seq, dim = input.shape
    scale16 = jnp.broadcast_to(scale_param, (16,))
    return _make_sc(batch, seq, dim, 0, seq)(input, pe[:seq], scale16)


# speedup vs baseline: 1.4345x; 1.4345x over previous
import functools
import jax
import jax.numpy as jnp
from jax import lax
from jax.experimental import pallas as pl
from jax.experimental.pallas import tpu as pltpu
from jax.experimental.pallas import tpu_sc as plsc

NC, NS = 2, 16
NW = NC * NS
R_SC = 16


def _make_sc(batch, rows_total, dim, row_offset, out_rows):
    rows_per_w = rows_total // NW
    nchunk = rows_per_w // R_SC
    assert nchunk % 2 == 0 and nchunk >= 2
    nvec = dim // 16
    mesh = plsc.VectorSubcoreMesh(core_axis_name="c", subcore_axis_name="s")

    @functools.partial(
        pl.kernel,
        mesh=mesh,
        out_type=jax.ShapeDtypeStruct((batch, out_rows, dim), jnp.float32),
        scratch_types=[
            pltpu.VMEM((16,), jnp.float32),
            pltpu.VMEM((R_SC, dim), jnp.float32),
            pltpu.VMEM((R_SC, dim), jnp.float32),
            pltpu.VMEM((R_SC, dim), jnp.float32),
            pltpu.VMEM((R_SC, dim), jnp.float32),
            pltpu.VMEM((R_SC, dim), jnp.float32),
            pltpu.VMEM((R_SC, dim), jnp.float32),
            pltpu.SemaphoreType.DMA((2,)),
            pltpu.SemaphoreType.DMA((2,)),
            pltpu.SemaphoreType.DMA((2,)),
        ],
    )
    def sc_fn(in_hbm, pe_hbm, scale_hbm, out_hbm,
              scale_v, pe_v0, pe_v1, in_v0, in_v1, out_v0, out_v1,
              pe_sem, in_sem, out_sem):
        wid = lax.axis_index("s") * NC + lax.axis_index("c")
        base = wid * rows_per_w
        pe_bufs = (pe_v0, pe_v1)
        in_bufs = (in_v0, in_v1)
        out_bufs = (out_v0, out_v1)

        def pe_copy(c, p):
            return pltpu.make_async_copy(
                pe_hbm.at[pl.ds(row_offset + base + c * R_SC, R_SC), :],
                pe_bufs[p], pe_sem.at[p])

        def in_copy(c, b, sl):
            return pltpu.make_async_copy(
                in_hbm.at[b, pl.ds(row_offset + base + c * R_SC, R_SC), :],
                in_bufs[sl], in_sem.at[sl])

        def out_copy(c, b, sl):
            return pltpu.make_async_copy(
                out_bufs[sl],
                out_hbm.at[b, pl.ds(base + c * R_SC, R_SC), :],
                out_sem.at[sl])

        pltpu.sync_copy(scale_hbm, scale_v)
        s = scale_v[...]

        pe_copy(0, 0).start()
        in_copy(0, 0, 0).start()

        def compute(in_v, pe_v, out_v):
            def row_body(r, _):
                for j in range(nvec):
                    sl = pl.ds(j * 16, 16)
                    out_v[r, sl] = in_v[r, sl] + pe_v[r, sl] * s
                return 0
            lax.fori_loop(0, R_SC, row_body, 0)

        def half(c2, half_idx):
            c = 2 * c2 + half_idx
            pe_copy(c, half_idx).wait()
            for b in range(batch):
                slot = b % 2
                in_copy(c, b, slot).wait()
                if b + 1 < batch:
                    in_copy(c, b + 1, (b + 1) % 2).start()
                elif half_idx == 0:
                    in_copy(c + 1, 0, 0).start()
                else:
                    @pl.when(c + 1 < nchunk)
                    def _():
                        in_copy(c + 1, 0, 0).start()
                if b == 0:
                    if half_idx == 0:
                        pe_copy(c + 1, 1).start()
                    else:
                        @pl.when(c + 1 < nchunk)
                        def _():
                            pe_copy(c + 1, 0).start()
                k = half_idx * batch + b
                if k >= 2:
                    out_copy(0, 0, slot).wait()
                else:
                    @pl.when(c2 > 0)
                    def _():
                        out_copy(0, 0, slot).wait()
                compute(in_bufs[slot], pe_bufs[half_idx], out_bufs[slot])
                out_copy(c, b, slot).start()

        def c2_body(c2, _):
            half(c2, 0)
            half(c2, 1)
            return 0

        lax.fori_loop(0, nchunk // 2, c2_body, 0)

        out_copy(0, 0, 0).wait()
        out_copy(0, 0, 1).wait()

    return sc_fn


R = 256
NBUF = 4
SC_ROWS = 1024


def _tc_body(scale_ref, in_hbm, pe_hbm, out_hbm,
             in_v, pe_v, out_v, in_sem, pe_sem, out_sem, *, tc_nchunk):
    s = scale_ref[0]

    def in_copy(j, slot):
        return pltpu.make_async_copy(
            in_hbm.at[:, pl.ds(j * R, R), :], in_v.at[slot], in_sem.at[slot])

    def pe_copy(j, slot):
        return pltpu.make_async_copy(
            pe_hbm.at[pl.ds(j * R, R), :], pe_v.at[slot], pe_sem.at[slot])

    def out_copy(j, slot):
        return pltpu.make_async_copy(
            out_v.at[slot], out_hbm.at[:, pl.ds(j * R, R), :], out_sem.at[slot])

    for k in range(NBUF):
        in_copy(k, k).start()
        pe_copy(k, k).start()

    def body(j, carry):
        slot = lax.rem(j, NBUF)
        in_copy(j, slot).wait()
        pe_copy(j, slot).wait()

        @pl.when(j >= NBUF)
        def _():
            out_copy(j - NBUF, slot).wait()

        out_v[slot] = in_v[slot] + s * pe_v[slot][None, :, :]
        out_copy(j, slot).start()

        nxt = j + NBUF

        @pl.when(nxt < tc_nchunk)
        def _():
            in_copy(nxt, slot).start()
            pe_copy(nxt, slot).start()

        return carry

    lax.fori_loop(0, tc_nchunk, body, 0)

    for k in range(NBUF):
        out_copy(tc_nchunk - NBUF + k, k).wait()


def kernel(input, pe, scale_param):
    batch, seq, dim = input.shape
    tc_rows = seq - SC_ROWS
    tc_nchunk = tc_rows // R

    tc_out = pl.pallas_call(
        functools.partial(_tc_body, tc_nchunk=tc_nchunk),
        in_specs=[
            pl.BlockSpec(memory_space=pltpu.SMEM),
            pl.BlockSpec(memory_space=pl.ANY),
            pl.BlockSpec(memory_space=pl.ANY),
        ],
        out_specs=pl.BlockSpec(memory_space=pl.ANY),
        out_shape=jax.ShapeDtypeStruct((batch, seq, dim), input.dtype),
        scratch_shapes=[
            pltpu.VMEM((NBUF, batch, R, dim), input.dtype),
            pltpu.VMEM((NBUF, R, dim), pe.dtype),
            pltpu.VMEM((NBUF, batch, R, dim), input.dtype),
            pltpu.SemaphoreType.DMA((NBUF,)),
            pltpu.SemaphoreType.DMA((NBUF,)),
            pltpu.SemaphoreType.DMA((NBUF,)),
        ],
    )(scale_param, input, pe[:seq])

    scale16 = jnp.broadcast_to(scale_param, (16,))
    sc_out = _make_sc(batch, SC_ROWS, dim, tc_rows, SC_ROWS)(
        input, pe[:seq], scale16)

    return lax.dynamic_update_slice(tc_out, sc_out, (0, tc_rows, 0))


# speedup vs baseline: 1.9861x; 1.3845x over previous
import jax
import jax.numpy as jnp
from jax.experimental import pallas as pl
from jax.experimental.pallas import tpu as pltpu


R = 128
NBUF = 16


def _pe_add_kernel(scale_ref, in_hbm, pe_hbm, out_hbm,
                   in_vmem, pe_vmem, out_vmem,
                   in_sem, pe_sem, out_sem):
    nchunk = in_hbm.shape[1] // R
    s = scale_ref[0]

    def in_copy(j, slot):
        return pltpu.make_async_copy(
            in_hbm.at[:, pl.ds(j * R, R), :], in_vmem.at[slot], in_sem.at[slot])

    def pe_copy(j, slot):
        return pltpu.make_async_copy(
            pe_hbm.at[pl.ds(j * R, R), :], pe_vmem.at[slot], pe_sem.at[slot])

    def out_copy(j, slot):
        return pltpu.make_async_copy(
            out_vmem.at[slot], out_hbm.at[:, pl.ds(j * R, R), :], out_sem.at[slot])

    for k in range(NBUF):
        in_copy(k, k).start()
        pe_copy(k, k).start()

    def body(j, carry):
        slot = jax.lax.rem(j, NBUF)
        in_copy(j, slot).wait()
        pe_copy(j, slot).wait()

        @pl.when(j >= NBUF)
        def _():
            out_copy(j - NBUF, slot).wait()

        out_vmem[slot] = in_vmem[slot] + s * pe_vmem[slot][None, :, :]
        out_copy(j, slot).start()

        nxt = j + NBUF

        @pl.when(nxt < nchunk)
        def _():
            in_copy(nxt, slot).start()
            pe_copy(nxt, slot).start()

        return carry

    jax.lax.fori_loop(0, nchunk, body, 0)

    for k in range(NBUF):
        out_copy(nchunk - NBUF + k, k).wait()


def kernel(input, pe, scale_param):
    batch, seq, dim = input.shape
    return pl.pallas_call(
        _pe_add_kernel,
        in_specs=[
            pl.BlockSpec(memory_space=pltpu.SMEM),
            pl.BlockSpec(memory_space=pl.ANY),
            pl.BlockSpec(memory_space=pl.ANY),
        ],
        out_specs=pl.BlockSpec(memory_space=pl.ANY),
        out_shape=jax.ShapeDtypeStruct((batch, seq, dim), input.dtype),
        scratch_shapes=[
            pltpu.VMEM((NBUF, batch, R, dim), input.dtype),
            pltpu.VMEM((NBUF, R, dim), pe.dtype),
            pltpu.VMEM((NBUF, batch, R, dim), input.dtype),
            pltpu.SemaphoreType.DMA((NBUF,)),
            pltpu.SemaphoreType.DMA((NBUF,)),
            pltpu.SemaphoreType.DMA((NBUF,)),
        ],
    )(scale_param, input, pe[:seq])


# speedup vs baseline: 1.9892x; 1.0015x over previous
import jax
import jax.numpy as jnp
from jax.experimental import pallas as pl
from jax.experimental.pallas import tpu as pltpu


R = 128
NBUF = 8


def _pe_add_kernel(scale_ref, in_hbm, pe_hbm, out_hbm,
                   in_vmem, pe_vmem, out_vmem,
                   in_sem, pe_sem, out_sem):
    nchunk = in_hbm.shape[1] // R
    s = scale_ref[0]

    def in_copy(j, slot):
        return pltpu.make_async_copy(
            in_hbm.at[:, pl.ds(j * R, R), :], in_vmem.at[slot], in_sem.at[slot])

    def pe_copy(j, slot):
        return pltpu.make_async_copy(
            pe_hbm.at[pl.ds(j * R, R), :], pe_vmem.at[slot], pe_sem.at[slot])

    def out_copy(j, slot):
        return pltpu.make_async_copy(
            out_vmem.at[slot], out_hbm.at[:, pl.ds(j * R, R), :], out_sem.at[slot])

    for k in range(NBUF):
        in_copy(k, k).start()
        pe_copy(k, k).start()

    def body(j, carry):
        slot = jax.lax.rem(j, NBUF)
        in_copy(j, slot).wait()
        pe_copy(j, slot).wait()

        @pl.when(j >= NBUF)
        def _():
            out_copy(j - NBUF, slot).wait()

        out_vmem[slot] = in_vmem[slot] + s * pe_vmem[slot][None, :, :]
        out_copy(j, slot).start()

        nxt = j + NBUF

        @pl.when(nxt < nchunk)
        def _():
            in_copy(nxt, slot).start()
            pe_copy(nxt, slot).start()

        return carry

    jax.lax.fori_loop(0, nchunk, body, 0)

    for k in range(NBUF):
        out_copy(nchunk - NBUF + k, k).wait()


def kernel(input, pe, scale_param):
    batch, seq, dim = input.shape
    return pl.pallas_call(
        _pe_add_kernel,
        in_specs=[
            pl.BlockSpec(memory_space=pltpu.SMEM),
            pl.BlockSpec(memory_space=pl.ANY),
            pl.BlockSpec(memory_space=pl.ANY),
        ],
        out_specs=pl.BlockSpec(memory_space=pl.ANY),
        out_shape=jax.ShapeDtypeStruct((batch, seq, dim), input.dtype),
        scratch_shapes=[
            pltpu.VMEM((NBUF, batch, R, dim), input.dtype),
            pltpu.VMEM((NBUF, R, dim), pe.dtype),
            pltpu.VMEM((NBUF, batch, R, dim), input.dtype),
            pltpu.SemaphoreType.DMA((NBUF,)),
            pltpu.SemaphoreType.DMA((NBUF,)),
            pltpu.SemaphoreType.DMA((NBUF,)),
        ],
    )(scale_param, input, pe[:seq])
